# baseline (speedup 1.0000x reference)
import functools

import jax
import jax.numpy as jnp
from jax import lax
from jax.experimental import pallas as pl
from jax.experimental.pallas import tpu as pltpu
from jax.experimental.pallas import tpu_sc as plsc

B = 16384
D = 64
NC = 2
NS = 16
NW = NC * NS
BPW = B // NW
ICH = 128
NCH = BPW // ICH
BLK = 16
NBLK = BPW // BLK


def _sc_body(f_hbm, w_hbm, t_hbm, b_hbm, out_hbm,
             idx_v, w_v, f_v, bias_v, part_v, sem):
    c = lax.axis_index("c")
    s = lax.axis_index("s")
    wid = s * NC + c
    base = wid * BPW

    pltpu.sync_copy(t_hbm.at[pl.ds(wid * NCH, NCH)], idx_v)

    copies = []
    for k in range(NCH):
        copies.append(pltpu.async_copy(w_hbm.at[idx_v.at[k]],
                                       w_v.at[pl.ds(k * ICH, ICH)], sem))
        copies.append(pltpu.async_copy(b_hbm.at[idx_v.at[k]],
                                       bias_v.at[pl.ds(k * ICH, ICH)], sem))
    copies.append(pltpu.async_copy(f_hbm.at[pl.ds(base, BPW)], f_v, sem))
    for cp in copies:
        cp.wait()

    lanes = lax.iota(jnp.int32, 16)

    def block_body(t, acc):
        rows = t * BLK + lanes

        def col_body(j, dacc):
            cols = jnp.full((16,), j, jnp.int32)
            wcol = plsc.load_gather(w_v, [rows, cols])
            fcol = plsc.load_gather(f_v, [rows, cols])
            return dacc + wcol * fcol

        dot = lax.fori_loop(0, D, col_body, jnp.zeros((16,), jnp.float32),
                            unroll=8)
        dvec = dot + bias_v[pl.ds(t * BLK, BLK)] - 1.0
        return acc + dvec * dvec

    acc = lax.fori_loop(0, NBLK, block_body, jnp.zeros((16,), jnp.float32))
    part_v[...] = acc
    pltpu.sync_copy(part_v, out_hbm.at[wid])


_sc_call = functools.partial(
    pl.kernel,
    out_type=jax.ShapeDtypeStruct((NW, 16), jnp.float32),
    mesh=plsc.VectorSubcoreMesh(core_axis_name="c", subcore_axis_name="s"),
    compiler_params=pltpu.CompilerParams(
        needs_layout_passes=False, use_tc_tiling_on_sc=False
    ),
    scratch_types=[
        pltpu.VMEM((NCH, ICH), jnp.int32),
        pltpu.VMEM((BPW, D), jnp.float32),
        pltpu.VMEM((BPW, D), jnp.float32),
        pltpu.VMEM((BPW,), jnp.float32),
        pltpu.VMEM((16,), jnp.float32),
        pltpu.SemaphoreType.DMA,
    ],
)(_sc_body)


def kernel(features, W, targets, b):
    t32 = targets.astype(jnp.int32).reshape(B // ICH, ICH)
    parts = _sc_call(features, W, t32, b)
    return jnp.sum(parts) * (0.5 / B)


# speedup vs baseline: 1.0268x; 1.0268x over previous
import functools

import jax
import jax.numpy as jnp
from jax import lax
from jax.experimental import pallas as pl
from jax.experimental.pallas import tpu as pltpu
from jax.experimental.pallas import tpu_sc as plsc

B = 16384
D = 64
NC = 2
NS = 16
NW = NC * NS
BPW = B // NW
ICH = 128
NCH = BPW // ICH
BLK = 16
NBLK = BPW // BLK


def _sc_body(ft_hbm, w2_hbm, t_hbm, b_hbm, out_hbm,
             idx_v, idx2_v, w_v, f_v, bias_v, part_v, sem):
    c = lax.axis_index("c")
    s = lax.axis_index("s")
    wid = s * NC + c
    base = wid * BPW

    pltpu.sync_copy(t_hbm.at[pl.ds(base, BPW)], idx_v)
    for i in range(BPW // BLK):
        sl = pl.ds(i * BLK, BLK)
        idx2_v[sl] = lax.shift_right_logical(idx_v[sl], 1)

    copies = []
    for k in range(NCH):
        sl = pl.ds(k * ICH, ICH)
        copies.append(pltpu.async_copy(w2_hbm.at[idx2_v.at[sl]],
                                       w_v.at[sl], sem))
        copies.append(pltpu.async_copy(b_hbm.at[idx_v.at[sl]],
                                       bias_v.at[sl], sem))
    copies.append(pltpu.async_copy(ft_hbm.at[:, pl.ds(base, BPW)], f_v, sem))
    for cp in copies:
        cp.wait()

    lanes = lax.iota(jnp.int32, BLK)

    def group_body(g, acc):
        col = g * BLK
        rows = col + lanes
        half = (idx_v[pl.ds(col, BLK)] & 1) * D
        dots = [jnp.zeros((BLK,), jnp.float32) for _ in range(4)]
        for j in range(D):
            wcol = plsc.load_gather(w_v, [rows, half + j])
            dots[j % 4] = dots[j % 4] + wcol * f_v[j, pl.ds(col, BLK)]
        dot = (dots[0] + dots[1]) + (dots[2] + dots[3])
        d = dot + bias_v[pl.ds(col, BLK)] - 1.0
        return acc + d * d

    acc = lax.fori_loop(0, NBLK, group_body, jnp.zeros((BLK,), jnp.float32))
    part_v[...] = acc
    pltpu.sync_copy(part_v, out_hbm.at[pl.ds(wid * BLK, BLK)])


_sc_call = functools.partial(
    pl.kernel,
    out_type=jax.ShapeDtypeStruct((NW * BLK,), jnp.float32),
    mesh=plsc.VectorSubcoreMesh(core_axis_name="c", subcore_axis_name="s"),
    compiler_params=pltpu.CompilerParams(
        needs_layout_passes=False, use_tc_tiling_on_sc=True
    ),
    scratch_types=[
        pltpu.VMEM((BPW,), jnp.int32),
        pltpu.VMEM((BPW,), jnp.int32),
        pltpu.VMEM((BPW, 2 * D), jnp.float32),
        pltpu.VMEM((D, BPW), jnp.float32),
        pltpu.VMEM((BPW,), jnp.float32),
        pltpu.VMEM((BLK,), jnp.float32),
        pltpu.SemaphoreType.DMA,
    ],
)(_sc_body)


def kernel(features, W, targets, b):
    t32 = targets.astype(jnp.int32)
    parts = _sc_call(features.T, W.reshape(-1, 2 * D), t32, b)
    return jnp.sum(parts) * (0.5 / B)


# speedup vs baseline: 1.1513x; 1.1212x over previous
import functools

import jax
import jax.numpy as jnp
from jax import lax
from jax.experimental import pallas as pl
from jax.experimental.pallas import tpu as pltpu
from jax.experimental.pallas import tpu_sc as plsc

B = 16384
D = 64
NC = 2
NS = 16
NW = NC * NS
BPW = B // NW
ICH = 128
NCH = BPW // ICH
BLK = 16
NBLK = BPW // BLK


def _sc_body(ft_hbm, w2_hbm, t_hbm, b_hbm, out_hbm,
             idx_v, w_v, f_v, bias_v, part_v, sem):
    c = lax.axis_index("c")
    s = lax.axis_index("s")
    wid = s * NC + c
    base = wid * BPW

    pltpu.sync_copy(t_hbm.at[pl.ds(base, BPW)], idx_v)

    copies = []
    for k in range(NCH):
        sl = pl.ds(k * ICH, ICH)
        copies.append(pltpu.async_copy(w2_hbm.at[idx_v.at[sl]],
                                       w_v.at[sl], sem))
        copies.append(pltpu.async_copy(b_hbm.at[idx_v.at[sl]],
                                       bias_v.at[sl], sem))
    copies.append(pltpu.async_copy(ft_hbm.at[:, pl.ds(base, BPW)], f_v, sem))
    for cp in copies:
        cp.wait()

    lanes = lax.iota(jnp.int32, BLK)

    def group_body(g, acc):
        col = g * BLK
        rows = col + lanes
        dots = [jnp.zeros((BLK,), jnp.float32) for _ in range(4)]
        for j in range(D):
            wcol = plsc.load_gather(w_v, [rows, jnp.full((BLK,), j, jnp.int32)])
            dots[j % 4] = dots[j % 4] + wcol * f_v[j, pl.ds(col, BLK)]
        dot = (dots[0] + dots[1]) + (dots[2] + dots[3])
        d = dot + bias_v[pl.ds(col, BLK)] - 1.0
        return acc + d * d

    acc = lax.fori_loop(0, NBLK, group_body, jnp.zeros((BLK,), jnp.float32))
    part_v[...] = acc
    pltpu.sync_copy(part_v, out_hbm.at[pl.ds(wid * BLK, BLK)])


_sc_call = functools.partial(
    pl.kernel,
    out_type=jax.ShapeDtypeStruct((NW * BLK,), jnp.float32),
    mesh=plsc.VectorSubcoreMesh(core_axis_name="c", subcore_axis_name="s"),
    compiler_params=pltpu.CompilerParams(
        needs_layout_passes=False, use_tc_tiling_on_sc=True
    ),
    scratch_types=[
        pltpu.VMEM((BPW,), jnp.int32),
        pltpu.VMEM((BPW, 2 * D), jnp.float32),
        pltpu.VMEM((D, BPW), jnp.float32),
        pltpu.VMEM((BPW,), jnp.float32),
        pltpu.VMEM((BLK,), jnp.float32),
        pltpu.SemaphoreType.DMA,
    ],
)(_sc_body)


def kernel(features, W, targets, b):
    t32 = targets.astype(jnp.int32)
    wp = jnp.pad(W, ((0, 0), (0, D)))
    parts = _sc_call(features.T, wp, t32, b)
    return jnp.sum(parts) * (0.5 / B)


# speedup vs baseline: 1.6046x; 1.3938x over previous
import functools

import jax
import jax.numpy as jnp
from jax import lax
from jax.experimental import pallas as pl
from jax.experimental.pallas import tpu as pltpu
from jax.experimental.pallas import tpu_sc as plsc

B = 16384
D = 64
NC = 2
NS = 16
NW = NC * NS
BPW = B // NW
ICH = 128
NCH = BPW // ICH
BLK = 16
NBLK = BPW // BLK
CCH = 64


def _sc_body(ft_hbm, w_hbm, t_hbm, b_hbm, out_hbm,
             idx_v, w_v, f_v, bias_v, part_v, sem, semw):
    c = lax.axis_index("c")
    s = lax.axis_index("s")
    wid = s * NC + c
    base = wid * BPW

    pltpu.sync_copy(t_hbm.at[pl.ds(base, BPW)], idx_v)

    copies = []
    for k in range(NCH):
        sl = pl.ds(k * ICH, ICH)
        copies.append(pltpu.async_copy(b_hbm.at[idx_v.at[sl]],
                                       bias_v.at[sl], sem))
    copies.append(pltpu.async_copy(ft_hbm.at[:, pl.ds(base, BPW)], f_v, sem))
    for cp in copies:
        cp.wait()

    lanes = lax.iota(jnp.int32, BLK)

    def chunk_body(ch, acc):
        cbase = ch * CCH
        for g in range(CCH // BLK):
            tv = idx_v[pl.ds(cbase + g * BLK, BLK)]
            for l in range(BLK):
                t = tv[l]
                rb = pl.multiple_of(t - (t & 7), 8)
                pltpu.async_copy(
                    w_hbm.at[pl.ds(rb, 8), :],
                    w_v.at[pl.ds((g * BLK + l) * 8, 8), :], semw)
        for _ in range(CCH):
            pltpu.make_async_copy(w_hbm.at[pl.ds(0, 8), :],
                                  w_v.at[pl.ds(0, 8), :], semw).wait()
        for g in range(CCH // BLK):
            col = cbase + g * BLK
            rows = (g * BLK + lanes) * 8 + (idx_v[pl.ds(col, BLK)] & 7)
            dots = [jnp.zeros((BLK,), jnp.float32) for _ in range(4)]
            for j in range(D):
                wcol = plsc.load_gather(
                    w_v, [rows, jnp.full((BLK,), j, jnp.int32)])
                dots[j % 4] = dots[j % 4] + wcol * f_v[j, pl.ds(col, BLK)]
            dot = (dots[0] + dots[1]) + (dots[2] + dots[3])
            d = dot + bias_v[pl.ds(col, BLK)] - 1.0
            acc = acc + d * d
        return acc

    acc = lax.fori_loop(0, BPW // CCH, chunk_body,
                        jnp.zeros((BLK,), jnp.float32))
    part_v[...] = acc
    pltpu.sync_copy(part_v, out_hbm.at[pl.ds(wid * BLK, BLK)])


_sc_call = functools.partial(
    pl.kernel,
    out_type=jax.ShapeDtypeStruct((NW * BLK,), jnp.float32),
    mesh=plsc.VectorSubcoreMesh(core_axis_name="c", subcore_axis_name="s"),
    compiler_params=pltpu.CompilerParams(
        needs_layout_passes=False, use_tc_tiling_on_sc=True
    ),
    scratch_types=[
        pltpu.VMEM((BPW,), jnp.int32),
        pltpu.VMEM((CCH * 8, D), jnp.float32),
        pltpu.VMEM((D, BPW), jnp.float32),
        pltpu.VMEM((BPW,), jnp.float32),
        pltpu.VMEM((BLK,), jnp.float32),
        pltpu.SemaphoreType.DMA,
        pltpu.SemaphoreType.DMA,
    ],
)(_sc_body)


def kernel(features, W, targets, b):
    t32 = targets.astype(jnp.int32)
    parts = _sc_call(features.T, W, t32, b)
    return jnp.sum(parts) * (0.5 / B)


# speedup vs baseline: 2.2947x; 1.4300x over previous
import functools

import jax
import jax.numpy as jnp
from jax import lax
from jax.experimental import pallas as pl
from jax.experimental.pallas import tpu as pltpu
from jax.experimental.pallas import tpu_sc as plsc

B = 16384
D = 64
NC = 2
NS = 16
NW = NC * NS
BPW = B // NW
ICH = 128
NCH = BPW // ICH
BLK = 16
NBLK = BPW // BLK
CCH = 64


def _sc_body(ft_hbm, w_hbm, t_hbm, b_hbm, out_hbm,
             idx_v, w_v, f_v, bias_v, part_v, sem, semw):
    c = lax.axis_index("c")
    s = lax.axis_index("s")
    wid = s * NC + c
    base = wid * BPW

    pltpu.sync_copy(t_hbm.at[pl.ds(base, BPW)], idx_v)

    copies = []
    for k in range(NCH):
        sl = pl.ds(k * ICH, ICH)
        copies.append(pltpu.async_copy(b_hbm.at[idx_v.at[sl]],
                                       bias_v.at[sl], sem))
    copies.append(pltpu.async_copy(ft_hbm.at[:, pl.ds(base, BPW)], f_v, sem))
    for cp in copies:
        cp.wait()

    lanes = lax.iota(jnp.int32, BLK)

    def chunk_body(ch, acc):
        cbase = ch * CCH
        for g in range(CCH // BLK):
            tv = idx_v[pl.ds(cbase + g * BLK, BLK)]
            for l in range(BLK):
                blk = lax.shift_right_logical(tv[l], 3)
                pltpu.async_copy(
                    w_hbm.at[pl.ds(blk, 1)],
                    w_v.at[pl.ds(g * BLK + l, 1)], semw)
        for _ in range(CCH):
            pltpu.make_async_copy(w_hbm.at[pl.ds(0, 1)],
                                  w_v.at[pl.ds(0, 1)], semw).wait()
        for g in range(CCH // BLK):
            col = cbase + g * BLK
            slot = g * BLK + lanes
            rsub = idx_v[pl.ds(col, BLK)] & 7
            dots = [jnp.zeros((BLK,), jnp.float32) for _ in range(4)]
            for j in range(D):
                wcol = plsc.load_gather(
                    w_v, [slot, rsub, jnp.full((BLK,), j, jnp.int32)])
                dots[j % 4] = dots[j % 4] + wcol * f_v[j, pl.ds(col, BLK)]
            dot = (dots[0] + dots[1]) + (dots[2] + dots[3])
            d = dot + bias_v[pl.ds(col, BLK)] - 1.0
            acc = acc + d * d
        return acc

    acc = lax.fori_loop(0, BPW // CCH, chunk_body,
                        jnp.zeros((BLK,), jnp.float32))
    part_v[...] = acc
    pltpu.sync_copy(part_v, out_hbm.at[pl.ds(wid * BLK, BLK)])


_sc_call = functools.partial(
    pl.kernel,
    out_type=jax.ShapeDtypeStruct((NW * BLK,), jnp.float32),
    mesh=plsc.VectorSubcoreMesh(core_axis_name="c", subcore_axis_name="s"),
    compiler_params=pltpu.CompilerParams(
        needs_layout_passes=False, use_tc_tiling_on_sc=True
    ),
    scratch_types=[
        pltpu.VMEM((BPW,), jnp.int32),
        pltpu.VMEM((CCH, 8, D), jnp.float32),
        pltpu.VMEM((D, BPW), jnp.float32),
        pltpu.VMEM((BPW,), jnp.float32),
        pltpu.VMEM((BLK,), jnp.float32),
        pltpu.SemaphoreType.DMA,
        pltpu.SemaphoreType.DMA,
    ],
)(_sc_body)


def kernel(features, W, targets, b):
    t32 = targets.astype(jnp.int32)
    parts = _sc_call(features.T, W.reshape(-1, 8, D), t32, b)
    return jnp.sum(parts) * (0.5 / B)


# speedup vs baseline: 2.4647x; 1.0741x over previous
import functools

import jax
import jax.numpy as jnp
from jax import lax
from jax.experimental import pallas as pl
from jax.experimental.pallas import tpu as pltpu
from jax.experimental.pallas import tpu_sc as plsc

B = 16384
D = 64
NC = 2
NS = 16
NW = NC * NS
BPW = B // NW
ICH = 128
NCH = BPW // ICH
BLK = 16
NBLK = BPW // BLK
CCH = 32


def _sc_body(ft_hbm, w_hbm, t_hbm, b_hbm, out_hbm,
             idx_v, w_v, w2_v, f_v, bias_v, part_v, sem, semw, semw2):
    c = lax.axis_index("c")
    s = lax.axis_index("s")
    wid = s * NC + c
    base = wid * BPW

    pltpu.sync_copy(t_hbm.at[pl.ds(base, BPW)], idx_v)

    copies = []
    for k in range(NCH):
        sl = pl.ds(k * ICH, ICH)
        copies.append(pltpu.async_copy(b_hbm.at[idx_v.at[sl]],
                                       bias_v.at[sl], sem))
    copies.append(pltpu.async_copy(ft_hbm.at[:, pl.ds(base, BPW)], f_v, sem))
    for cp in copies:
        cp.wait()

    lanes = lax.iota(jnp.int32, BLK)
    NCHK = BPW // CCH

    def fire(ch, buf, semx):
        for g in range(CCH // BLK):
            tv = idx_v[pl.ds(ch * CCH + g * BLK, BLK)]
            for l in range(BLK):
                blk = lax.shift_right_logical(tv[l], 3)
                pltpu.async_copy(w_hbm.at[pl.ds(blk, 1)],
                                 buf.at[pl.ds(g * BLK + l, 1)], semx)

    def drain_compute(ch, buf, semx):
        for _ in range(CCH):
            pltpu.make_async_copy(w_hbm.at[pl.ds(0, 1)],
                                  buf.at[pl.ds(0, 1)], semx).wait()
        for g in range(CCH // BLK):
            col = ch * CCH + g * BLK
            slot = g * BLK + lanes
            rsub = idx_v[pl.ds(col, BLK)] & 7
            dots = [jnp.zeros((BLK,), jnp.float32) for _ in range(4)]
            for j in range(D):
                wcol = plsc.load_gather(
                    buf, [slot, rsub, jnp.full((BLK,), j, jnp.int32)])
                dots[j % 4] = dots[j % 4] + wcol * f_v[j, pl.ds(col, BLK)]
            dot = (dots[0] + dots[1]) + (dots[2] + dots[3])
            d = dot + bias_v[pl.ds(col, BLK)] - 1.0
            part_v[...] = part_v[...] + d * d

    part_v[...] = jnp.zeros((BLK,), jnp.float32)
    fire(0, w_v, semw)

    def chunk_body(ch, carry):
        nxt = ch + 1

        @pl.when(jnp.logical_and(nxt < NCHK, (nxt & 1) == 0))
        def _():
            fire(nxt, w_v, semw)

        @pl.when(jnp.logical_and(nxt < NCHK, (nxt & 1) == 1))
        def _():
            fire(nxt, w2_v, semw2)

        @pl.when((ch & 1) == 0)
        def _():
            drain_compute(ch, w_v, semw)

        @pl.when((ch & 1) == 1)
        def _():
            drain_compute(ch, w2_v, semw2)

        return carry

    lax.fori_loop(0, NCHK, chunk_body, 0)
    pltpu.sync_copy(part_v, out_hbm.at[pl.ds(wid * BLK, BLK)])


_sc_call = functools.partial(
    pl.kernel,
    out_type=jax.ShapeDtypeStruct((NW * BLK,), jnp.float32),
    mesh=plsc.VectorSubcoreMesh(core_axis_name="c", subcore_axis_name="s"),
    compiler_params=pltpu.CompilerParams(
        needs_layout_passes=False, use_tc_tiling_on_sc=True
    ),
    scratch_types=[
        pltpu.VMEM((BPW,), jnp.int32),
        pltpu.VMEM((CCH, 8, D), jnp.float32),
        pltpu.VMEM((CCH, 8, D), jnp.float32),
        pltpu.VMEM((D, BPW), jnp.float32),
        pltpu.VMEM((BPW,), jnp.float32),
        pltpu.VMEM((BLK,), jnp.float32),
        pltpu.SemaphoreType.DMA,
        pltpu.SemaphoreType.DMA,
        pltpu.SemaphoreType.DMA,
    ],
)(_sc_body)


def kernel(features, W, targets, b):
    t32 = targets.astype(jnp.int32)
    parts = _sc_call(features.T, W.reshape(-1, 8, D), t32, b)
    return jnp.sum(parts) * (0.5 / B)
